# Initial kernel scaffold; baseline (speedup 1.0000x reference)
import jax
import jax.numpy as jnp
from jax.experimental import pallas as pl


def kernel(edge_index, emb, Wl, bl, Wr, gamma, beta, jkW, jkb):
    raise NotImplementedError("write your pallas kernel here")


# speedup vs baseline: 6.1082x; 6.1082x over previous
import functools

import jax
import jax.numpy as jnp
from jax import lax
from jax.experimental import pallas as pl
from jax.experimental.pallas import tpu as pltpu
from jax.experimental.pallas import tpu_sc as plsc

N = 10000
H = 128
E = 320000

NC = 2
NS = 16
NW = NC * NS
EPW = E // NW
CH = 128
FULL = EPW // CH
TAIL = EPW - FULL * CH
NP = 10240
RPT = NP // NS

_MESH = plsc.VectorSubcoreMesh(core_axis_name="c", subcore_axis_name="s")


def _sc_agg_body(with_deg, *refs):
    if with_deg:
        (x_hbm, src_hbm, dst_hbm, zeros_hbm, part_hbm, deg_hbm,
         idx_src, idx_dst, idx_src_t, idx_dst_t, rows, ones, degbuf,
         acc, dacc, sem) = refs
    else:
        (x_hbm, src_hbm, dst_hbm, zeros_hbm, part_hbm,
         idx_src, idx_dst, idx_src_t, idx_dst_t, rows,
         acc, sem) = refs

    c = lax.axis_index("c")
    s = lax.axis_index("s")
    base = s * RPT

    pltpu.sync_copy(zeros_hbm, rows)
    for k in range(RPT // CH):
        pltpu.sync_copy(rows, acc.at[pl.ds(base + k * CH, CH)])
    if with_deg:
        for k in range(RPT // 16):
            degbuf[pl.ds(k * 16, 16)] = jnp.zeros((16,), jnp.float32)
        pltpu.sync_copy(degbuf, dacc.at[pl.ds(base, RPT)])
        for k in range(CH // 16):
            ones[pl.ds(k * 16, 16)] = jnp.ones((16,), jnp.float32)
    plsc.subcore_barrier()

    ebase = (c * NS + s) * EPW

    def chunk(i, _):
        off = ebase + i * CH
        pltpu.sync_copy(src_hbm.at[pl.ds(off, CH)], idx_src)
        pltpu.sync_copy(dst_hbm.at[pl.ds(off, CH)], idx_dst)
        pltpu.async_copy(x_hbm.at[idx_src], rows, sem).wait()
        pltpu.sync_copy(rows, acc.at[idx_dst], add=True)
        if with_deg:
            pltpu.sync_copy(ones, dacc.at[idx_dst], add=True)
        return _

    lax.fori_loop(0, FULL, chunk, None)

    toff = ebase + FULL * CH
    pltpu.sync_copy(src_hbm.at[pl.ds(toff, TAIL)], idx_src_t)
    pltpu.sync_copy(dst_hbm.at[pl.ds(toff, TAIL)], idx_dst_t)
    pltpu.async_copy(x_hbm.at[idx_src_t], rows.at[pl.ds(0, TAIL)], sem).wait()
    pltpu.sync_copy(rows.at[pl.ds(0, TAIL)], acc.at[idx_dst_t], add=True)
    if with_deg:
        pltpu.sync_copy(ones.at[pl.ds(0, TAIL)], dacc.at[idx_dst_t], add=True)

    plsc.subcore_barrier()

    for k in range(RPT // CH):
        r0 = base + k * CH
        pltpu.sync_copy(acc.at[pl.ds(r0, CH)], rows)
        pltpu.sync_copy(rows, part_hbm.at[c, pl.ds(r0, CH)])
    if with_deg:
        pltpu.sync_copy(dacc.at[pl.ds(base, RPT)], degbuf)
        pltpu.sync_copy(degbuf, deg_hbm.at[c, pl.ds(base, RPT)])


def _make_sc_agg(with_deg):
    part_t = jax.ShapeDtypeStruct((NC, NP, H), jnp.float32)
    out_type = [part_t]
    scratch = [
        pltpu.VMEM((CH,), jnp.int32),
        pltpu.VMEM((CH,), jnp.int32),
        pltpu.VMEM((TAIL,), jnp.int32),
        pltpu.VMEM((TAIL,), jnp.int32),
        pltpu.VMEM((CH, H), jnp.float32),
    ]
    if with_deg:
        out_type.append(jax.ShapeDtypeStruct((NC, NP), jnp.float32))
        scratch += [
            pltpu.VMEM((CH,), jnp.float32),
            pltpu.VMEM((RPT,), jnp.float32),
        ]
    scratch.append(pltpu.VMEM_SHARED((NP, H), jnp.float32))
    if with_deg:
        scratch.append(pltpu.VMEM_SHARED((NP,), jnp.float32))
    scratch.append(pltpu.SemaphoreType.DMA)
    return pl.kernel(
        functools.partial(_sc_agg_body, with_deg),
        out_type=out_type if with_deg else part_t,
        mesh=_MESH,
        scratch_types=scratch,
    )


_sc_agg_deg = _make_sc_agg(True)
_sc_agg = _make_sc_agg(False)

R = 400


def _tc_layer_body(residual, part_ref, deg_ref, x_ref, wl_ref, bl_ref,
                   wr_ref, g_ref, b_ref, o_ref):
    p = part_ref[0] + part_ref[1]
    deg = deg_ref[0] + deg_ref[1]
    agg = p * (1.0 / jnp.maximum(deg, 1.0))
    x = x_ref[...]
    h = (jnp.dot(agg, wl_ref[...], preferred_element_type=jnp.float32,
                 precision=lax.Precision.HIGHEST)
         + bl_ref[...]
         + jnp.dot(x, wr_ref[...], preferred_element_type=jnp.float32,
                   precision=lax.Precision.HIGHEST))
    mu = jnp.mean(h, axis=-1, keepdims=True)
    var = jnp.mean((h - mu) ** 2, axis=-1, keepdims=True)
    h = (h - mu) / jnp.sqrt(var + 1e-5) * g_ref[...] + b_ref[...]
    h = jnp.maximum(h, 0.0)
    if residual:
        h = h + x
    o_ref[...] = h


def _tc_final_body(part_ref, deg_ref, x2_ref, x1_ref, x0_ref, wl_ref, bl_ref,
                   wr_ref, g_ref, b_ref, jkw_ref, jkb_ref, o_ref):
    p = part_ref[0] + part_ref[1]
    deg = deg_ref[0] + deg_ref[1]
    agg = p * (1.0 / jnp.maximum(deg, 1.0))
    x2 = x2_ref[...]
    h = (jnp.dot(agg, wl_ref[...], preferred_element_type=jnp.float32,
                 precision=lax.Precision.HIGHEST)
         + bl_ref[...]
         + jnp.dot(x2, wr_ref[...], preferred_element_type=jnp.float32,
                   precision=lax.Precision.HIGHEST))
    mu = jnp.mean(h, axis=-1, keepdims=True)
    var = jnp.mean((h - mu) ** 2, axis=-1, keepdims=True)
    h = (h - mu) / jnp.sqrt(var + 1e-5) * g_ref[...] + b_ref[...]
    x3 = jnp.maximum(h, 0.0) + x2
    jkw = jkw_ref[...]
    out = jkb_ref[...]
    for i, xi in enumerate((x0_ref[...], x1_ref[...], x2, x3)):
        out = out + jnp.dot(xi, jkw[i * H:(i + 1) * H, :],
                            preferred_element_type=jnp.float32,
                            precision=lax.Precision.HIGHEST)
    o_ref[...] = out


def _row_specs():
    return [
        pl.BlockSpec((NC, R, H), lambda i: (0, i, 0)),
        pl.BlockSpec((NC, R, 1), lambda i: (0, i, 0)),
        pl.BlockSpec((R, H), lambda i: (i, 0)),
    ]


_W_SPEC = pl.BlockSpec((H, H), lambda i: (0, 0))
_V_SPEC = pl.BlockSpec((1, H), lambda i: (0, 0))
_O_SPEC = pl.BlockSpec((R, H), lambda i: (i, 0))
_O_SHAPE = jax.ShapeDtypeStruct((N, H), jnp.float32)


def _make_tc_layer(residual):
    return pl.pallas_call(
        functools.partial(_tc_layer_body, residual),
        grid=(N // R,),
        in_specs=_row_specs() + [_W_SPEC, _V_SPEC, _W_SPEC, _V_SPEC, _V_SPEC],
        out_specs=_O_SPEC,
        out_shape=_O_SHAPE,
    )


_tc_layer0 = _make_tc_layer(False)
_tc_layer1 = _make_tc_layer(True)

_tc_final = pl.pallas_call(
    _tc_final_body,
    grid=(N // R,),
    in_specs=(_row_specs()
              + [pl.BlockSpec((R, H), lambda i: (i, 0))] * 2
              + [_W_SPEC, _V_SPEC, _W_SPEC, _V_SPEC, _V_SPEC,
                 pl.BlockSpec((4 * H, H), lambda i: (0, 0)), _V_SPEC]),
    out_specs=_O_SPEC,
    out_shape=_O_SHAPE,
)


def kernel(edge_index, emb, Wl, bl, Wr, gamma, beta, jkW, jkb):
    src = edge_index[0]
    dst = edge_index[1]
    zeros = jnp.zeros((CH, H), jnp.float32)
    v = lambda a: a.reshape(1, H)

    part1, deg = _sc_agg_deg(emb, src, dst, zeros)
    deg = deg.reshape(NC, NP, 1)
    x1 = _tc_layer0(part1, deg, emb, Wl[0], v(bl[0]), Wr[0],
                    v(gamma[0]), v(beta[0]))
    part2 = _sc_agg(x1, src, dst, zeros)
    x2 = _tc_layer1(part2, deg, x1, Wl[1], v(bl[1]), Wr[1],
                    v(gamma[1]), v(beta[1]))
    part3 = _sc_agg(x2, src, dst, zeros)
    out = _tc_final(part3, deg, x2, x1, emb, Wl[2], v(bl[2]), Wr[2],
                    v(gamma[2]), v(beta[2]), jkW, v(jkb))
    return out
